# baseline (speedup 1.0000x reference)
import functools

import jax
import jax.numpy as jnp
from jax import lax
from jax.experimental import pallas as pl
from jax.experimental.pallas import tpu as pltpu
from jax.experimental.pallas import tpu_sc as plsc


def kernel(x, relative_position_bias_table, rpe_index):
    B, H, N, N2 = x.shape
    V = relative_position_bias_table.shape[0]
    L = 16
    C = 512

    tbl_flat = jnp.reshape(
        jnp.transpose(relative_position_bias_table), (-1,))
    idx = rpe_index.astype(jnp.int32)

    info = plsc.get_sparse_core_info()
    nw = info.num_cores * info.num_subcores
    rows_per_w = N // nw

    mesh = plsc.VectorSubcoreMesh(core_axis_name="c", subcore_axis_name="s")

    @functools.partial(
        pl.kernel,
        mesh=mesh,
        out_type=jax.ShapeDtypeStruct((B, H, N, N2), jnp.float32),
        compiler_params=pltpu.CompilerParams(needs_layout_passes=False),
        scratch_types=[
            pltpu.VMEM((H * V,), jnp.float32),
            pltpu.VMEM((C,), jnp.int32),
            pltpu.VMEM((C,), jnp.int32),
            pltpu.VMEM((B, H, C), jnp.float32),
            pltpu.VMEM((B, H, C), jnp.float32),
            pltpu.VMEM((B, H, C), jnp.float32),
            pltpu.VMEM((B, H, C), jnp.float32),
            pltpu.SemaphoreType.DMA,
            pltpu.SemaphoreType.DMA,
            pltpu.SemaphoreType.DMA,
            pltpu.SemaphoreType.DMA,
            pltpu.SemaphoreType.DMA,
            pltpu.SemaphoreType.DMA,
        ],
    )
    def run(x_hbm, tbl_hbm, idx_hbm, out_hbm,
            tbl_v, idx0, idx1, xin0, xin1, xo0, xo1,
            si0, si1, sx0, sx1, so0, so1):
        wid = lax.axis_index("s") * info.num_cores + lax.axis_index("c")
        base = wid * rows_per_w
        pltpu.sync_copy(tbl_hbm, tbl_v)

        idxb = (idx0, idx1)
        xinb = (xin0, xin1)
        xob = (xo0, xo1)
        sib = (si0, si1)
        sxb = (sx0, sx1)
        sob = (so0, so1)

        def issue_in(row, k):
            j0 = k * C
            pltpu.async_copy(
                idx_hbm.at[row, pl.ds(j0, C)], idxb[k], sib[k])
            pltpu.async_copy(
                x_hbm.at[:, :, row, pl.ds(j0, C)], xinb[k], sxb[k])

        def substep(r, k):
            row = base + r
            j0 = k * C
            pltpu.make_async_copy(
                idx_hbm.at[row, pl.ds(j0, C)], idxb[k], sib[k]).wait()
            pltpu.make_async_copy(
                x_hbm.at[:, :, row, pl.ds(j0, C)], xinb[k], sxb[k]).wait()

            @pl.when(r >= 1)
            def _():
                pltpu.make_async_copy(
                    xob[k], out_hbm.at[:, :, row - 1, pl.ds(j0, C)],
                    sob[k]).wait()

            @plsc.parallel_loop(0, C // L, 1, unroll=4)
            def _(v):
                start = pl.multiple_of(v * L, L)
                iv = idxb[k][pl.ds(start, L)]
                for h in range(H):
                    bias = plsc.load_gather(tbl_v, [iv + h * V])
                    for b in range(B):
                        xob[k][b, h, pl.ds(start, L)] = (
                            xinb[k][b, h, pl.ds(start, L)] + bias)

            pltpu.async_copy(
                xob[k], out_hbm.at[:, :, row, pl.ds(j0, C)], sob[k])

            @pl.when(r + 1 < rows_per_w)
            def _():
                issue_in(row + 1, k)

        issue_in(base, 0)
        issue_in(base, 1)

        def row_body(r, carry):
            substep(r, 0)
            substep(r, 1)
            return carry

        lax.fori_loop(0, rows_per_w, row_body, 0)

        last = base + rows_per_w - 1
        for k in range(2):
            pltpu.make_async_copy(
                xob[k], out_hbm.at[:, :, last, pl.ds(k * C, C)],
                sob[k]).wait()

    return run(x, tbl_flat, idx)


# speedup vs baseline: 1.1153x; 1.1153x over previous
import functools

import jax
import jax.numpy as jnp
from jax import lax
from jax.experimental import pallas as pl
from jax.experimental.pallas import tpu as pltpu
from jax.experimental.pallas import tpu_sc as plsc


def kernel(x, relative_position_bias_table, rpe_index):
    B, H, N, N2 = x.shape
    V = relative_position_bias_table.shape[0]
    L = 16
    C = 512

    tbl_flat = jnp.reshape(
        jnp.transpose(relative_position_bias_table), (-1,))
    idx = rpe_index.astype(jnp.int32)

    info = plsc.get_sparse_core_info()
    nw = info.num_cores * info.num_subcores
    rows_per_w = N // nw

    mesh = plsc.VectorSubcoreMesh(core_axis_name="c", subcore_axis_name="s")

    @functools.partial(
        pl.kernel,
        mesh=mesh,
        out_type=jax.ShapeDtypeStruct((B, H, N, N2), jnp.float32),
        compiler_params=pltpu.CompilerParams(needs_layout_passes=False),
        scratch_types=[
            pltpu.VMEM((H * V,), jnp.float32),
            pltpu.VMEM((C,), jnp.int32),
            pltpu.VMEM((C,), jnp.int32),
            pltpu.VMEM((B, H, C), jnp.float32),
            pltpu.VMEM((B, H, C), jnp.float32),
            pltpu.VMEM((B, H, C), jnp.float32),
            pltpu.VMEM((B, H, C), jnp.float32),
            pltpu.SemaphoreType.DMA,
            pltpu.SemaphoreType.DMA,
            pltpu.SemaphoreType.DMA,
            pltpu.SemaphoreType.DMA,
            pltpu.SemaphoreType.DMA,
            pltpu.SemaphoreType.DMA,
        ],
    )
    def run(x_hbm, tbl_hbm, idx_hbm, out_hbm,
            tbl_v, idx0, idx1, xin0, xin1, xo0, xo1,
            si0, si1, sx0, sx1, so0, so1):
        wid = lax.axis_index("s") * info.num_cores + lax.axis_index("c")
        base = wid * rows_per_w
        pltpu.sync_copy(tbl_hbm, tbl_v)

        idxb = (idx0, idx1)
        xinb = (xin0, xin1)
        xob = (xo0, xo1)
        sib = (si0, si1)
        sxb = (sx0, sx1)
        sob = (so0, so1)

        def issue_in(row, k):
            j0 = k * C
            pltpu.async_copy(
                idx_hbm.at[row, pl.ds(j0, C)], idxb[k], sib[k])
            pltpu.async_copy(
                x_hbm.at[:, :, row, pl.ds(j0, C)], xinb[k], sxb[k])

        def substep(r, k):
            row = base + r
            j0 = k * C
            pltpu.make_async_copy(
                idx_hbm.at[row, pl.ds(j0, C)], idxb[k], sib[k]).wait()
            pltpu.make_async_copy(
                x_hbm.at[:, :, row, pl.ds(j0, C)], xinb[k], sxb[k]).wait()

            @pl.when(r >= 1)
            def _():
                pltpu.make_async_copy(
                    xob[k], out_hbm.at[:, :, row - 1, pl.ds(j0, C)],
                    sob[k]).wait()

            @plsc.parallel_loop(0, C // L, 1, unroll=3)
            def _(v):
                start = pl.multiple_of(v * L, L)
                iv = idxb[k][pl.ds(start, L)]
                for h in range(H):
                    bias = plsc.load_gather(tbl_v, [iv + h * V])
                    for b in range(B):
                        xob[k][b, h, pl.ds(start, L)] = (
                            xinb[k][b, h, pl.ds(start, L)] + bias)

            pltpu.async_copy(
                xob[k], out_hbm.at[:, :, row, pl.ds(j0, C)], sob[k])

            @pl.when(r + 1 < rows_per_w)
            def _():
                issue_in(row + 1, k)

        issue_in(base, 0)
        issue_in(base, 1)

        def row_body(r, carry):
            substep(r, 0)
            substep(r, 1)
            return carry

        lax.fori_loop(0, rows_per_w, row_body, 0)

        last = base + rows_per_w - 1
        for k in range(2):
            pltpu.make_async_copy(
                xob[k], out_hbm.at[:, :, last, pl.ds(k * C, C)],
                sob[k]).wait()

    return run(x, tbl_flat, idx)


# speedup vs baseline: 1.1757x; 1.0541x over previous
import functools

import jax
import jax.numpy as jnp
from jax import lax
from jax.experimental import pallas as pl
from jax.experimental.pallas import tpu as pltpu
from jax.experimental.pallas import tpu_sc as plsc


def kernel(x, relative_position_bias_table, rpe_index):
    B, H, N, N2 = x.shape
    V = relative_position_bias_table.shape[0]
    L = 16
    C = 512

    tbl_flat = jnp.reshape(
        jnp.transpose(relative_position_bias_table), (-1,))
    idx = rpe_index.astype(jnp.int32)

    info = plsc.get_sparse_core_info()
    nw = info.num_cores * info.num_subcores
    rows_per_w = N // nw
    P = rows_per_w // 2

    mesh = plsc.VectorSubcoreMesh(core_axis_name="c", subcore_axis_name="s")

    @functools.partial(
        pl.kernel,
        mesh=mesh,
        out_type=jax.ShapeDtypeStruct((B, H, N, N2), jnp.float32),
        compiler_params=pltpu.CompilerParams(needs_layout_passes=False),
        scratch_types=(
            [pltpu.VMEM((H * V,), jnp.float32)]
            + [pltpu.VMEM((B, H, C), jnp.float32)] * 4
            + [pltpu.VMEM((C,), jnp.int32)] * 2
            + [pltpu.SemaphoreType.DMA] * 10
        ),
    )
    def run(x_hbm, tbl_hbm, idx_hbm, out_hbm,
            tbl_v, xb0, xb1, xb2, xb3, ib0, ib1,
            sx0, sx1, sx2, sx3, so0, so1, so2, so3, si0, si1):
        wid = lax.axis_index("s") * info.num_cores + lax.axis_index("c")
        base = wid * rows_per_w
        pltpu.sync_copy(tbl_hbm, tbl_v)

        xb = (xb0, xb1, xb2, xb3)
        sx = (sx0, sx1, sx2, sx3)
        so = (so0, so1, so2, so3)
        ib = (ib0, ib1)
        si = (si0, si1)

        def issue_x(row, j0, k):
            pltpu.async_copy(
                x_hbm.at[:, :, row, pl.ds(j0, C)], xb[k], sx[k])

        def issue_idx(row, j0, k):
            pltpu.async_copy(idx_hbm.at[row, pl.ds(j0, C)], ib[k], si[k])

        def substep(p, s):
            row = base + 2 * p + (s // 2)
            j0 = (s % 2) * C
            kx = s
            ki = s % 2

            def wait_prev_out():
                pltpu.make_async_copy(
                    xb[(s + 2) % 4],
                    out_hbm.at[:, :, row - 1, pl.ds(j0, C)],
                    so[(s + 2) % 4]).wait()

            if s >= 2:
                wait_prev_out()
            else:
                @pl.when(p >= 1)
                def _():
                    wait_prev_out()

            if s < 2:
                issue_x(row + 1, j0, (s + 2) % 4)
            else:
                @pl.when(p < P - 1)
                def _():
                    issue_x(row + 1, j0, (s + 2) % 4)

            nrow = row if s in (0, 2) else row + 1
            nj0 = C if s in (0, 2) else 0
            if s == 3:
                @pl.when(p < P - 1)
                def _():
                    issue_idx(nrow, nj0, (s + 1) % 2)
            else:
                issue_idx(nrow, nj0, (s + 1) % 2)

            pltpu.make_async_copy(
                idx_hbm.at[row, pl.ds(j0, C)], ib[ki], si[ki]).wait()
            pltpu.make_async_copy(
                x_hbm.at[:, :, row, pl.ds(j0, C)], xb[kx], sx[kx]).wait()

            @plsc.parallel_loop(0, C // L, 1, unroll=2)
            def _(v):
                start = pl.multiple_of(v * L, L)
                iv = ib[ki][pl.ds(start, L)]
                for h in range(H):
                    bias = plsc.load_gather(tbl_v, [iv + h * V])
                    for b in range(B):
                        plsc.addupdate(
                            xb[kx].at[b, h, pl.ds(start, L)], bias)

            pltpu.async_copy(
                xb[kx], out_hbm.at[:, :, row, pl.ds(j0, C)], so[kx])

        issue_x(base, 0, 0)
        issue_x(base, C, 1)
        issue_idx(base, 0, 0)

        def group_body(p, carry):
            for s in range(4):
                substep(p, s)
            return carry

        lax.fori_loop(0, P, group_body, 0)

        last = base + rows_per_w - 1
        for k, j0 in ((2, 0), (3, C)):
            pltpu.make_async_copy(
                xb[k], out_hbm.at[:, :, last, pl.ds(j0, C)], so[k]).wait()

    return run(x, tbl_flat, idx)


# speedup vs baseline: 1.1887x; 1.0111x over previous
import functools

import jax
import jax.numpy as jnp
from jax import lax
from jax.experimental import pallas as pl
from jax.experimental.pallas import tpu as pltpu
from jax.experimental.pallas import tpu_sc as plsc


def kernel(x, relative_position_bias_table, rpe_index):
    B, H, N, N2 = x.shape
    V = relative_position_bias_table.shape[0]
    L = 16
    C = 512

    tbl_flat = jnp.reshape(
        jnp.transpose(relative_position_bias_table), (-1,))
    idx = rpe_index.astype(jnp.int32)

    info = plsc.get_sparse_core_info()
    nw = info.num_cores * info.num_subcores
    rows_per_w = N // nw
    P = rows_per_w // 2

    mesh = plsc.VectorSubcoreMesh(core_axis_name="c", subcore_axis_name="s")

    @functools.partial(
        pl.kernel,
        mesh=mesh,
        out_type=jax.ShapeDtypeStruct((B, H, N, N2), jnp.float32),
        compiler_params=pltpu.CompilerParams(needs_layout_passes=False),
        scratch_types=(
            [pltpu.VMEM((H * V,), jnp.float32)]
            + [pltpu.VMEM((B, H, C), jnp.float32)] * 4
            + [pltpu.VMEM((C,), jnp.int32)] * 2
            + [pltpu.SemaphoreType.DMA] * 10
        ),
    )
    def run(x_hbm, tbl_hbm, idx_hbm, out_hbm,
            tbl_v, xb0, xb1, xb2, xb3, ib0, ib1,
            sx0, sx1, sx2, sx3, so0, so1, so2, so3, si0, si1):
        wid = lax.axis_index("s") * info.num_cores + lax.axis_index("c")
        base = wid * rows_per_w

        xb = (xb0, xb1, xb2, xb3)
        sx = (sx0, sx1, sx2, sx3)
        so = (so0, so1, so2, so3)
        ib = (ib0, ib1)
        si = (si0, si1)

        def issue_x(row, j0, k):
            pltpu.async_copy(
                x_hbm.at[:, :, row, pl.ds(j0, C)], xb[k], sx[k])

        def issue_idx(row, j0, k):
            pltpu.async_copy(idx_hbm.at[row, pl.ds(j0, C)], ib[k], si[k])

        def substep(p, s):
            row = base + 2 * p + (s // 2)
            j0 = (s % 2) * C
            kx = s
            ki = s % 2

            def wait_prev_out():
                pltpu.make_async_copy(
                    xb[(s + 2) % 4],
                    out_hbm.at[:, :, row - 1, pl.ds(j0, C)],
                    so[(s + 2) % 4]).wait()

            if s >= 2:
                wait_prev_out()
            else:
                @pl.when(p >= 1)
                def _():
                    wait_prev_out()

            if s < 2:
                issue_x(row + 1, j0, (s + 2) % 4)
            else:
                @pl.when(p < P - 1)
                def _():
                    issue_x(row + 1, j0, (s + 2) % 4)

            nrow = row if s in (0, 2) else row + 1
            nj0 = C if s in (0, 2) else 0
            if s == 3:
                @pl.when(p < P - 1)
                def _():
                    issue_idx(nrow, nj0, (s + 1) % 2)
            else:
                issue_idx(nrow, nj0, (s + 1) % 2)

            pltpu.make_async_copy(
                idx_hbm.at[row, pl.ds(j0, C)], ib[ki], si[ki]).wait()
            pltpu.make_async_copy(
                x_hbm.at[:, :, row, pl.ds(j0, C)], xb[kx], sx[kx]).wait()

            @plsc.parallel_loop(0, C // L, 1, unroll=2)
            def _(v):
                start = pl.multiple_of(v * L, L)
                iv = ib[ki][pl.ds(start, L)]
                for h in range(H):
                    bias = plsc.load_gather(tbl_v, [iv + h * V])
                    for b in range(B):
                        plsc.addupdate(
                            xb[kx].at[b, h, pl.ds(start, L)], bias)

            pltpu.async_copy(
                xb[kx], out_hbm.at[:, :, row, pl.ds(j0, C)], so[kx])

        issue_x(base, 0, 0)
        issue_x(base, C, 1)
        issue_idx(base, 0, 0)
        pltpu.sync_copy(tbl_hbm, tbl_v)

        def group_body(p, carry):
            for s in range(4):
                substep(p, s)
            return carry

        lax.fori_loop(0, P, group_body, 0)

        last = base + rows_per_w - 1
        for k, j0 in ((2, 0), (3, C)):
            pltpu.make_async_copy(
                xb[k], out_hbm.at[:, :, last, pl.ds(j0, C)], so[k]).wait()

    return run(x, tbl_flat, idx)


# speedup vs baseline: 1.2116x; 1.0193x over previous
import functools

import jax
import jax.numpy as jnp
from jax import lax
from jax.experimental import pallas as pl
from jax.experimental.pallas import tpu as pltpu
from jax.experimental.pallas import tpu_sc as plsc


def kernel(x, relative_position_bias_table, rpe_index):
    B, H, N, N2 = x.shape
    V = relative_position_bias_table.shape[0]
    L = 16
    C = 512

    tbl_flat = jnp.reshape(
        jnp.transpose(relative_position_bias_table), (-1,))
    idx = rpe_index.astype(jnp.int32)

    info = plsc.get_sparse_core_info()
    nw = info.num_cores * info.num_subcores
    rows_per_w = N // nw
    P = rows_per_w // 2

    mesh = plsc.VectorSubcoreMesh(core_axis_name="c", subcore_axis_name="s")

    @functools.partial(
        pl.kernel,
        mesh=mesh,
        out_type=jax.ShapeDtypeStruct((B, H, N, N2), jnp.float32),
        compiler_params=pltpu.CompilerParams(needs_layout_passes=False),
        scratch_types=(
            [pltpu.VMEM((H * V,), jnp.float32)]
            + [pltpu.VMEM((B, H, C), jnp.float32)] * 4
            + [pltpu.VMEM((C,), jnp.int32)] * 2
            + [pltpu.SemaphoreType.DMA] * 10
        ),
    )
    def run(x_hbm, tbl_hbm, idx_hbm, out_hbm,
            tbl_v, xb0, xb1, xb2, xb3, ib0, ib1,
            sx0, sx1, sx2, sx3, so0, so1, so2, so3, si0, si1):
        wid = lax.axis_index("s") * info.num_cores + lax.axis_index("c")
        base = wid * rows_per_w

        xb = (xb0, xb1, xb2, xb3)
        sx = (sx0, sx1, sx2, sx3)
        so = (so0, so1, so2, so3)
        ib = (ib0, ib1)
        si = (si0, si1)

        def issue_x(row, j0, k):
            pltpu.async_copy(
                x_hbm.at[:, :, row, pl.ds(j0, C)], xb[k], sx[k])

        def issue_idx(row, j0, k):
            pltpu.async_copy(idx_hbm.at[row, pl.ds(j0, C)], ib[k], si[k])

        def substep(p, s):
            row = base + 2 * p + (s // 2)
            j0 = (s % 2) * C
            kx = s
            ki = s % 2

            def wait_prev_out():
                pltpu.make_async_copy(
                    xb[(s + 2) % 4],
                    out_hbm.at[:, :, row - 1, pl.ds(j0, C)],
                    so[(s + 2) % 4]).wait()

            if s >= 2:
                wait_prev_out()
            else:
                @pl.when(p >= 1)
                def _():
                    wait_prev_out()

            if s < 2:
                issue_x(row + 1, j0, (s + 2) % 4)
            else:
                @pl.when(p < P - 1)
                def _():
                    issue_x(row + 1, j0, (s + 2) % 4)

            nrow = row if s in (0, 2) else row + 1
            nj0 = C if s in (0, 2) else 0
            if s == 3:
                @pl.when(p < P - 1)
                def _():
                    issue_idx(nrow, nj0, (s + 1) % 2)
            else:
                issue_idx(nrow, nj0, (s + 1) % 2)

            pltpu.make_async_copy(
                idx_hbm.at[row, pl.ds(j0, C)], ib[ki], si[ki]).wait()
            pltpu.make_async_copy(
                x_hbm.at[:, :, row, pl.ds(j0, C)], xb[kx], sx[kx]).wait()

            @plsc.parallel_loop(0, C // L, 1, unroll=1)
            def _(v):
                start = pl.multiple_of(v * L, L)
                iv = ib[ki][pl.ds(start, L)]
                for h in range(H):
                    bias = plsc.load_gather(tbl_v, [iv + h * V])
                    for b in range(B):
                        plsc.addupdate(
                            xb[kx].at[b, h, pl.ds(start, L)], bias)

            pltpu.async_copy(
                xb[kx], out_hbm.at[:, :, row, pl.ds(j0, C)], so[kx])

        issue_x(base, 0, 0)
        issue_x(base, C, 1)
        issue_idx(base, 0, 0)
        pltpu.sync_copy(tbl_hbm, tbl_v)

        def group_body(p, carry):
            for s in range(4):
                substep(p, s)
            return carry

        lax.fori_loop(0, P, group_body, 0)

        last = base + rows_per_w - 1
        for k, j0 in ((2, 0), (3, C)):
            pltpu.make_async_copy(
                xb[k], out_hbm.at[:, :, last, pl.ds(j0, C)], so[k]).wait()

    return run(x, tbl_flat, idx)
